# Initial kernel scaffold; baseline (speedup 1.0000x reference)
import jax
import jax.numpy as jnp
from jax.experimental import pallas as pl


def kernel(input_features, centroid_w, ln_g, ln_b, ff1_w, ff1_b, ff2_w, ff2_b):
    raise NotImplementedError("write your pallas kernel here")


# speedup vs baseline: 1.0050x; 1.0050x over previous
import functools

import jax
import jax.numpy as jnp
from jax.experimental import pallas as pl
from jax.experimental.pallas import tpu as pltpu

_LN_EPS = 1e-5


def _fused_mlp_body(x_ref, g_ref, b_ref, w1_ref, b1_ref, w2_ref, b2_ref,
                    o_ref):
    x = x_ref[...]
    mu = jnp.mean(x, axis=1, keepdims=True)
    xc = x - mu
    var = jnp.mean(xc * xc, axis=1, keepdims=True)
    h = xc * jax.lax.rsqrt(var + _LN_EPS) * g_ref[...] + b_ref[...]
    h1 = jnp.dot(h.astype(jnp.bfloat16), w1_ref[...],
                 preferred_element_type=jnp.float32) + b1_ref[...]
    h1 = jnp.maximum(h1, 0.0)
    h2 = jnp.dot(h1.astype(jnp.bfloat16), w2_ref[...],
                 preferred_element_type=jnp.float32) + b2_ref[...]
    o_ref[...] = x + h2


@functools.partial(jax.jit, static_argnames=("block_rows",))
def _fused_moe(x, ln_g, ln_b, w1t, ff1_b, w2t, ff2_b, block_rows=512):
    n, d = x.shape
    ff = w1t.shape[1]
    grid = (n // block_rows,)
    return pl.pallas_call(
        _fused_mlp_body,
        grid=grid,
        in_specs=[
            pl.BlockSpec((block_rows, d), lambda i: (i, 0)),
            pl.BlockSpec((1, d), lambda i: (0, 0)),
            pl.BlockSpec((1, d), lambda i: (0, 0)),
            pl.BlockSpec((d, ff), lambda i: (0, 0)),
            pl.BlockSpec((1, ff), lambda i: (0, 0)),
            pl.BlockSpec((ff, d), lambda i: (0, 0)),
            pl.BlockSpec((1, d), lambda i: (0, 0)),
        ],
        out_specs=pl.BlockSpec((block_rows, d), lambda i: (i, 0)),
        out_shape=jax.ShapeDtypeStruct((n, d), jnp.float32),
        compiler_params=pltpu.CompilerParams(
            dimension_semantics=("arbitrary",),
        ),
    )(x, ln_g, ln_b, w1t, ff1_b, w2t, ff2_b)


def kernel(input_features, centroid_w, ln_g, ln_b, ff1_w, ff1_b, ff2_w,
           ff2_b):
    del centroid_w
    s, b, d = input_features.shape
    ff = ff1_w.shape[0]
    x = input_features.reshape(s * b, d)
    out = _fused_moe(
        x,
        ln_g.reshape(1, d),
        ln_b.reshape(1, d),
        ff1_w.T.astype(jnp.bfloat16),
        ff1_b.reshape(1, ff),
        ff2_w.T.astype(jnp.bfloat16),
        ff2_b.reshape(1, d),
    )
    return out.reshape(s, b, d)


# speedup vs baseline: 1.0055x; 1.0005x over previous
import functools

import jax
import jax.numpy as jnp
from jax.experimental import pallas as pl
from jax.experimental.pallas import tpu as pltpu

_LN_EPS = 1e-5


def _fused_mlp_body(x_ref, g_ref, b_ref, w1_ref, b1_ref, w2_ref, b2_ref,
                    o_ref):
    x = x_ref[...]
    mu = jnp.mean(x, axis=1, keepdims=True)
    xc = x - mu
    var = jnp.mean(xc * xc, axis=1, keepdims=True)
    h = xc * jax.lax.rsqrt(var + _LN_EPS) * g_ref[...] + b_ref[...]
    h1 = jnp.dot(h.astype(jnp.bfloat16), w1_ref[...],
                 preferred_element_type=jnp.float32) + b1_ref[...]
    h1 = jnp.maximum(h1, 0.0)
    h2 = jnp.dot(h1.astype(jnp.bfloat16), w2_ref[...],
                 preferred_element_type=jnp.float32) + b2_ref[...]
    o_ref[...] = x + h2


@functools.partial(jax.jit, static_argnames=("block_rows",))
def _fused_moe(x, ln_g, ln_b, w1t, ff1_b, w2t, ff2_b, block_rows=512):
    n, d = x.shape
    ff = w1t.shape[1]
    grid = (n // block_rows,)
    return pl.pallas_call(
        _fused_mlp_body,
        grid=grid,
        in_specs=[
            pl.BlockSpec((block_rows, d), lambda i: (i, 0)),
            pl.BlockSpec((1, d), lambda i: (0, 0)),
            pl.BlockSpec((1, d), lambda i: (0, 0)),
            pl.BlockSpec((d, ff), lambda i: (0, 0)),
            pl.BlockSpec((1, ff), lambda i: (0, 0)),
            pl.BlockSpec((ff, d), lambda i: (0, 0)),
            pl.BlockSpec((1, d), lambda i: (0, 0)),
        ],
        out_specs=pl.BlockSpec((block_rows, d), lambda i: (i, 0)),
        out_shape=jax.ShapeDtypeStruct((n, d), jnp.float32),
        compiler_params=pltpu.CompilerParams(
            dimension_semantics=("parallel",),
        ),
    )(x, ln_g, ln_b, w1t, ff1_b, w2t, ff2_b)


def kernel(input_features, centroid_w, ln_g, ln_b, ff1_w, ff1_b, ff2_w,
           ff2_b):
    del centroid_w
    s, b, d = input_features.shape
    ff = ff1_w.shape[0]
    x = input_features.reshape(s * b, d)
    out = _fused_moe(
        x,
        ln_g.reshape(1, d),
        ln_b.reshape(1, d),
        ff1_w.T.astype(jnp.bfloat16),
        ff1_b.reshape(1, ff),
        ff2_w.T.astype(jnp.bfloat16),
        ff2_b.reshape(1, d),
    )
    return out.reshape(s, b, d)


# speedup vs baseline: 2.1579x; 2.1460x over previous
import functools

import jax
import jax.numpy as jnp
from jax.experimental import pallas as pl
from jax.experimental.pallas import tpu as pltpu

_LN_EPS = 1e-5


def _fused_mlp_body(x_ref, g_ref, b_ref, w1_ref, b1_ref, w2_ref, b2_ref,
                    o_ref):
    x3 = x_ref[...]
    t, bb, d = x3.shape
    x = x3.reshape(t * bb, d)
    mu = jnp.mean(x, axis=1, keepdims=True)
    xc = x - mu
    var = jnp.mean(xc * xc, axis=1, keepdims=True)
    h = xc * jax.lax.rsqrt(var + _LN_EPS) * g_ref[...] + b_ref[...]
    h1 = jnp.dot(h.astype(jnp.bfloat16), w1_ref[...],
                 preferred_element_type=jnp.float32) + b1_ref[...]
    h1 = jnp.maximum(h1, 0.0)
    h2 = jnp.dot(h1.astype(jnp.bfloat16), w2_ref[...],
                 preferred_element_type=jnp.float32) + b2_ref[...]
    o_ref[...] = (x + h2).reshape(t, bb, d)


@functools.partial(jax.jit, static_argnames=("block_rows",))
def _fused_moe(x, ln_g, ln_b, w1t, ff1_b, w2t, ff2_b, block_rows=256):
    s, b, d = x.shape
    ff = w1t.shape[1]
    grid = (s // block_rows,)
    return pl.pallas_call(
        _fused_mlp_body,
        grid=grid,
        in_specs=[
            pl.BlockSpec((block_rows, b, d), lambda i: (i, 0, 0)),
            pl.BlockSpec((1, d), lambda i: (0, 0)),
            pl.BlockSpec((1, d), lambda i: (0, 0)),
            pl.BlockSpec((d, ff), lambda i: (0, 0)),
            pl.BlockSpec((1, ff), lambda i: (0, 0)),
            pl.BlockSpec((ff, d), lambda i: (0, 0)),
            pl.BlockSpec((1, d), lambda i: (0, 0)),
        ],
        out_specs=pl.BlockSpec((block_rows, b, d), lambda i: (i, 0, 0)),
        out_shape=jax.ShapeDtypeStruct((s, b, d), jnp.float32),
        compiler_params=pltpu.CompilerParams(
            dimension_semantics=("parallel",),
        ),
    )(x, ln_g, ln_b, w1t, ff1_b, w2t, ff2_b)


def kernel(input_features, centroid_w, ln_g, ln_b, ff1_w, ff1_b, ff2_w,
           ff2_b):
    del centroid_w
    s, b, d = input_features.shape
    ff = ff1_w.shape[0]
    out = _fused_moe(
        input_features,
        ln_g.reshape(1, d),
        ln_b.reshape(1, d),
        ff1_w.T.astype(jnp.bfloat16),
        ff1_b.reshape(1, ff),
        ff2_w.T.astype(jnp.bfloat16),
        ff2_b.reshape(1, d),
    )
    return out


# speedup vs baseline: 2.1809x; 1.0107x over previous
import functools

import jax
import jax.numpy as jnp
from jax.experimental import pallas as pl
from jax.experimental.pallas import tpu as pltpu

_LN_EPS = 1e-5


def _fused_mlp_body(x_ref, g_ref, b_ref, w1_ref, b1_ref, w2_ref, b2_ref,
                    o_ref):
    x3 = x_ref[...]
    t, bb, d = x3.shape
    x = x3.reshape(t * bb, d)
    mu = jnp.mean(x, axis=1, keepdims=True)
    xc = x - mu
    var = jnp.mean(xc * xc, axis=1, keepdims=True)
    h = xc * jax.lax.rsqrt(var + _LN_EPS) * g_ref[...] + b_ref[...]
    h1 = jnp.dot(h.astype(jnp.bfloat16), w1_ref[...],
                 preferred_element_type=jnp.float32) + b1_ref[...]
    h1 = jnp.maximum(h1, 0.0)
    h2 = jnp.dot(h1.astype(jnp.bfloat16), w2_ref[...],
                 preferred_element_type=jnp.float32) + b2_ref[...]
    o_ref[...] = (x + h2).reshape(t, bb, d)


@functools.partial(jax.jit, static_argnames=("block_rows",))
def _fused_moe(x, ln_g, ln_b, w1t, ff1_b, w2t, ff2_b, block_rows=512):
    s, b, d = x.shape
    ff = w1t.shape[1]
    grid = (s // block_rows,)
    return pl.pallas_call(
        _fused_mlp_body,
        grid=grid,
        in_specs=[
            pl.BlockSpec((block_rows, b, d), lambda i: (i, 0, 0)),
            pl.BlockSpec((1, d), lambda i: (0, 0)),
            pl.BlockSpec((1, d), lambda i: (0, 0)),
            pl.BlockSpec((d, ff), lambda i: (0, 0)),
            pl.BlockSpec((1, ff), lambda i: (0, 0)),
            pl.BlockSpec((ff, d), lambda i: (0, 0)),
            pl.BlockSpec((1, d), lambda i: (0, 0)),
        ],
        out_specs=pl.BlockSpec((block_rows, b, d), lambda i: (i, 0, 0)),
        out_shape=jax.ShapeDtypeStruct((s, b, d), jnp.float32),
        compiler_params=pltpu.CompilerParams(
            dimension_semantics=("parallel",),
        ),
    )(x, ln_g, ln_b, w1t, ff1_b, w2t, ff2_b)


def kernel(input_features, centroid_w, ln_g, ln_b, ff1_w, ff1_b, ff2_w,
           ff2_b):
    del centroid_w
    s, b, d = input_features.shape
    ff = ff1_w.shape[0]
    out = _fused_moe(
        input_features,
        ln_g.reshape(1, d),
        ln_b.reshape(1, d),
        ff1_w.T.astype(jnp.bfloat16),
        ff1_b.reshape(1, ff),
        ff2_w.T.astype(jnp.bfloat16),
        ff2_b.reshape(1, d),
    )
    return out


# speedup vs baseline: 2.2268x; 1.0211x over previous
import functools

import jax
import jax.numpy as jnp
from jax.experimental import pallas as pl
from jax.experimental.pallas import tpu as pltpu

_LN_EPS = 1e-5


_CHUNKS = 2


def _fused_mlp_body(x_ref, g_ref, b_ref, w1_ref, b1_ref, w2_ref, b2_ref,
                    o_ref):
    t, bb, d = x_ref.shape
    rows = t * bb
    c = rows // _CHUNKS
    xf = x_ref.reshape(rows, d)
    of = o_ref.reshape(rows, d)
    g = g_ref[...]
    be = b_ref[...]
    w1 = w1_ref[...]
    b1 = b1_ref[...]
    w2 = w2_ref[...]
    b2 = b2_ref[...]

    def process(x):
        mu = jnp.mean(x, axis=1, keepdims=True)
        xc = x - mu
        var = jnp.mean(xc * xc, axis=1, keepdims=True)
        h = xc * jax.lax.rsqrt(var + _LN_EPS) * g + be
        h1 = jnp.dot(h.astype(jnp.bfloat16), w1,
                     preferred_element_type=jnp.float32) + b1
        h1 = jnp.maximum(h1, 0.0)
        h2 = jnp.dot(h1.astype(jnp.bfloat16), w2,
                     preferred_element_type=jnp.float32) + b2
        return x + h2

    ys = [process(xf[pl.ds(k * c, c), :]) for k in range(_CHUNKS)]
    for k in range(_CHUNKS):
        of[pl.ds(k * c, c), :] = ys[k]


@functools.partial(jax.jit, static_argnames=("block_rows",))
def _fused_moe(x, ln_g, ln_b, w1t, ff1_b, w2t, ff2_b, block_rows=512):
    s, b, d = x.shape
    ff = w1t.shape[1]
    grid = (s // block_rows,)
    return pl.pallas_call(
        _fused_mlp_body,
        grid=grid,
        in_specs=[
            pl.BlockSpec((block_rows, b, d), lambda i: (i, 0, 0)),
            pl.BlockSpec((1, d), lambda i: (0, 0)),
            pl.BlockSpec((1, d), lambda i: (0, 0)),
            pl.BlockSpec((d, ff), lambda i: (0, 0)),
            pl.BlockSpec((1, ff), lambda i: (0, 0)),
            pl.BlockSpec((ff, d), lambda i: (0, 0)),
            pl.BlockSpec((1, d), lambda i: (0, 0)),
        ],
        out_specs=pl.BlockSpec((block_rows, b, d), lambda i: (i, 0, 0)),
        out_shape=jax.ShapeDtypeStruct((s, b, d), jnp.float32),
        compiler_params=pltpu.CompilerParams(
            dimension_semantics=("parallel",),
        ),
    )(x, ln_g, ln_b, w1t, ff1_b, w2t, ff2_b)


def kernel(input_features, centroid_w, ln_g, ln_b, ff1_w, ff1_b, ff2_w,
           ff2_b):
    del centroid_w
    s, b, d = input_features.shape
    ff = ff1_w.shape[0]
    out = _fused_moe(
        input_features,
        ln_g.reshape(1, d),
        ln_b.reshape(1, d),
        ff1_w.T.astype(jnp.bfloat16),
        ff1_b.reshape(1, ff),
        ff2_w.T.astype(jnp.bfloat16),
        ff2_b.reshape(1, d),
    )
    return out
